# baseline (speedup 1.0000x reference)
import jax
import jax.numpy as jnp
from jax.experimental import pallas as pl
from jax.experimental.pallas import tpu as pltpu

B = 16384
D = 64
H = 8
C = 512
TAU = 0.3
BLK = 8192
K = 4


def _fused_body(z_ref, hot_ref, gap_ref, wq_ref, bq_ref, wf_ref, bf_ref,
                mcat_ref, wk_ref, wo_ref, ckt_ref, cv_ref, out_ref):
    z = z_ref[...]
    x = z * hot_ref[...]
    query = jnp.dot(x, wq_ref[...], preferred_element_type=jnp.float32) + bq_ref[...]

    wkm = jnp.dot(wk_ref[...], mcat_ref[...], preferred_element_type=jnp.float32)
    wcmb = jnp.dot(wkm, wo_ref[...], preferred_element_type=jnp.float32) * (1.0 / H)
    v_mem = jnp.dot(query, wcmb, preferred_element_type=jnp.float32)

    qn = query / jnp.maximum(
        jnp.sqrt(jnp.sum(query * query, axis=1, keepdims=True)), 1e-8)
    ckt = ckt_ref[...]
    cn = ckt / jnp.maximum(
        jnp.sqrt(jnp.sum(ckt * ckt, axis=0, keepdims=True)), 1e-8)
    sims = jnp.dot(qn, cn, preferred_element_type=jnp.float32)

    t = jnp.max(sims, axis=1, keepdims=True)
    for _ in range(K - 1):
        t = jnp.max(jnp.where(sims < t, sims, -jnp.inf),
                    axis=1, keepdims=True)
    w = (sims >= t).astype(jnp.float32)
    v_ep = jnp.dot(w, cv_ref[...], preferred_element_type=jnp.float32) * (1.0 / K)

    v_c = (v_mem + v_ep) * 0.5
    wf = wf_ref[...]
    fused = (jnp.dot(z, wf[:D], preferred_element_type=jnp.float32)
             + jnp.dot(v_c, wf[D:], preferred_element_type=jnp.float32)
             + bf_ref[...])
    active = gap_ref[...] > TAU
    out_ref[...] = jnp.where(active, fused, z)


def kernel(z, E, hot_dims, gap_norm, W_q, b_q, W_f, b_f, mem_M, W_k, W_o,
           cache_keys, cache_vals):
    del E
    gap2d = gap_norm.reshape(B, 1)
    mcat = mem_M.reshape(H * D, D)
    ckt = cache_keys.T
    bq2d = b_q.reshape(1, D)
    bf2d = b_f.reshape(1, D)

    row_spec = pl.BlockSpec((BLK, D), lambda i: (i, 0))
    gap_spec = pl.BlockSpec((BLK, 1), lambda i: (i, 0))

    def full(shape):
        return pl.BlockSpec(shape, lambda i: tuple(0 for _ in shape))

    z_enriched = pl.pallas_call(
        _fused_body,
        grid=(B // BLK,),
        in_specs=[
            row_spec,
            row_spec,
            gap_spec,
            full((D, D)),
            full((1, D)),
            full((2 * D, D)),
            full((1, D)),
            full((H * D, D)),
            full((D, H * D)),
            full((D, D)),
            full((D, C)),
            full((C, D)),
        ],
        out_specs=row_spec,
        out_shape=jax.ShapeDtypeStruct((B, D), jnp.float32),
        compiler_params=pltpu.CompilerParams(
            dimension_semantics=("arbitrary",)),
    )(z, hot_dims, gap2d, W_q, bq2d, W_f, bf2d, mcat, W_k, W_o, ckt,
      cache_vals)

    cf_loss = jnp.zeros((), dtype=jnp.float32)
    return z_enriched, cf_loss


# speedup vs baseline: 1.0388x; 1.0388x over previous
import jax
import jax.numpy as jnp
from jax.experimental import pallas as pl
from jax.experimental.pallas import tpu as pltpu

B = 16384
D = 64
H = 8
C = 512
TAU = 0.3
BLK = 4096
K = 4


def _fused_body(z_ref, hot_ref, gap_ref, wq_ref, bq_ref, wf_ref, bf_ref,
                mcat_ref, wk_ref, wo_ref, ckt_ref, cv_ref, out_ref):
    z = z_ref[...]
    x = z * hot_ref[...]
    query = jnp.dot(x, wq_ref[...], preferred_element_type=jnp.float32) + bq_ref[...]

    wkm = jnp.dot(wk_ref[...], mcat_ref[...], preferred_element_type=jnp.float32)
    wcmb = jnp.dot(wkm, wo_ref[...], preferred_element_type=jnp.float32) * (1.0 / H)
    v_mem = jnp.dot(query, wcmb, preferred_element_type=jnp.float32)

    qn = query / jnp.maximum(
        jnp.sqrt(jnp.sum(query * query, axis=1, keepdims=True)), 1e-8)
    ckt = ckt_ref[...]
    cn = ckt / jnp.maximum(
        jnp.sqrt(jnp.sum(ckt * ckt, axis=0, keepdims=True)), 1e-8)
    sims = jnp.dot(qn, cn, preferred_element_type=jnp.float32)

    t = jnp.max(sims, axis=1, keepdims=True)
    for _ in range(K - 1):
        t = jnp.max(jnp.where(sims < t, sims, -jnp.inf),
                    axis=1, keepdims=True)
    w = (sims >= t).astype(jnp.float32)
    v_ep = jnp.dot(w, cv_ref[...], preferred_element_type=jnp.float32) * (1.0 / K)

    v_c = (v_mem + v_ep) * 0.5
    wf = wf_ref[...]
    fused = (jnp.dot(z, wf[:D], preferred_element_type=jnp.float32)
             + jnp.dot(v_c, wf[D:], preferred_element_type=jnp.float32)
             + bf_ref[...])
    active = gap_ref[...] > TAU
    out_ref[...] = jnp.where(active, fused, z)


def kernel(z, E, hot_dims, gap_norm, W_q, b_q, W_f, b_f, mem_M, W_k, W_o,
           cache_keys, cache_vals):
    del E
    gap2d = gap_norm.reshape(B, 1)
    mcat = mem_M.reshape(H * D, D)
    ckt = cache_keys.T
    bq2d = b_q.reshape(1, D)
    bf2d = b_f.reshape(1, D)

    row_spec = pl.BlockSpec((BLK, D), lambda i: (i, 0))
    gap_spec = pl.BlockSpec((BLK, 1), lambda i: (i, 0))

    def full(shape):
        return pl.BlockSpec(shape, lambda i: tuple(0 for _ in shape))

    z_enriched = pl.pallas_call(
        _fused_body,
        grid=(B // BLK,),
        in_specs=[
            row_spec,
            row_spec,
            gap_spec,
            full((D, D)),
            full((1, D)),
            full((2 * D, D)),
            full((1, D)),
            full((H * D, D)),
            full((D, H * D)),
            full((D, D)),
            full((D, C)),
            full((C, D)),
        ],
        out_specs=row_spec,
        out_shape=jax.ShapeDtypeStruct((B, D), jnp.float32),
        compiler_params=pltpu.CompilerParams(
            dimension_semantics=("arbitrary",)),
    )(z, hot_dims, gap2d, W_q, bq2d, W_f, bf2d, mcat, W_k, W_o, ckt,
      cache_vals)

    cf_loss = jnp.zeros((), dtype=jnp.float32)
    return z_enriched, cf_loss


# speedup vs baseline: 1.0397x; 1.0009x over previous
import jax
import jax.numpy as jnp
from jax.experimental import pallas as pl
from jax.experimental.pallas import tpu as pltpu

B = 16384
D = 64
H = 8
C = 512
TAU = 0.3
BLK = 4096
K = 4


def _fused_body(z_ref, hot_ref, gap_ref, wq_ref, bq_ref, wf_ref, bf_ref,
                mcat_ref, wk_ref, wo_ref, ckt_ref, cv_ref, out_ref):
    z = z_ref[...]
    x = z * hot_ref[...]
    query = jnp.dot(x, wq_ref[...], preferred_element_type=jnp.float32) + bq_ref[...]

    wkm = jnp.dot(wk_ref[...], mcat_ref[...], preferred_element_type=jnp.float32)
    wcmb = jnp.dot(wkm, wo_ref[...], preferred_element_type=jnp.float32) * (1.0 / H)
    v_mem = jnp.dot(query, wcmb, preferred_element_type=jnp.float32)

    qn = query / jnp.maximum(
        jnp.sqrt(jnp.sum(query * query, axis=1, keepdims=True)), 1e-8)
    ckt = ckt_ref[...]
    cn = ckt / jnp.maximum(
        jnp.sqrt(jnp.sum(ckt * ckt, axis=0, keepdims=True)), 1e-8)
    sims = jnp.dot(qn, cn, preferred_element_type=jnp.float32)

    t = jnp.max(sims, axis=1, keepdims=True)
    for _ in range(K - 1):
        t = jnp.max(jnp.where(sims < t, sims, -jnp.inf),
                    axis=1, keepdims=True)
    w = (sims >= t).astype(jnp.float32)
    v_ep = jnp.dot(w, cv_ref[...], preferred_element_type=jnp.float32) * (1.0 / K)

    v_c = (v_mem + v_ep) * 0.5
    wf = wf_ref[...]
    fused = (jnp.dot(z, wf[:D], preferred_element_type=jnp.float32)
             + jnp.dot(v_c, wf[D:], preferred_element_type=jnp.float32)
             + bf_ref[...])
    active = gap_ref[...] > TAU
    out_ref[...] = jnp.where(active, fused, z)


def kernel(z, E, hot_dims, gap_norm, W_q, b_q, W_f, b_f, mem_M, W_k, W_o,
           cache_keys, cache_vals):
    del E
    gap2d = gap_norm.reshape(B, 1)
    mcat = mem_M.reshape(H * D, D)
    ckt = cache_keys.T
    bq2d = b_q.reshape(1, D)
    bf2d = b_f.reshape(1, D)

    row_spec = pl.BlockSpec((BLK, D), lambda i: (i, 0))
    gap_spec = pl.BlockSpec((BLK, 1), lambda i: (i, 0))

    def full(shape):
        return pl.BlockSpec(shape, lambda i: tuple(0 for _ in shape))

    z_enriched = pl.pallas_call(
        _fused_body,
        grid=(B // BLK,),
        in_specs=[
            row_spec,
            row_spec,
            gap_spec,
            full((D, D)),
            full((1, D)),
            full((2 * D, D)),
            full((1, D)),
            full((H * D, D)),
            full((D, H * D)),
            full((D, D)),
            full((D, C)),
            full((C, D)),
        ],
        out_specs=row_spec,
        out_shape=jax.ShapeDtypeStruct((B, D), jnp.float32),
        compiler_params=pltpu.CompilerParams(
            dimension_semantics=("parallel",)),
    )(z, hot_dims, gap2d, W_q, bq2d, W_f, bf2d, mcat, W_k, W_o, ckt,
      cache_vals)

    cf_loss = jnp.zeros((), dtype=jnp.float32)
    return z_enriched, cf_loss
